# baseline (speedup 1.0000x reference)
import functools

import jax
import jax.numpy as jnp
from jax import lax
from jax.experimental import pallas as pl
from jax.experimental.pallas import tpu as pltpu
from jax.experimental.pallas import tpu_sc as plsc

B, L = 16384, 200
VOCAB, DIM = 10, 3
NC, NS = 2, 16
NW = NC * NS
RPW = B // NW
CH = 128
NCHUNK = RPW // CH
EPC = CH * L
VPC = EPC // 16
MAGIC = 5243

_mesh = plsc.VectorSubcoreMesh(core_axis_name="c", subcore_axis_name="s")


@functools.partial(
    pl.kernel,
    mesh=_mesh,
    out_type=jax.ShapeDtypeStruct((B * DIM,), jnp.float32),
    scratch_types=[
        pltpu.VMEM((2, 2, EPC), jnp.int32),
        pltpu.VMEM((2 * CH * 16,), jnp.float32),
        pltpu.VMEM((2 * VOCAB * DIM, 16), jnp.float32),
        pltpu.VMEM((2 * CH * DIM,), jnp.float32),
        pltpu.SemaphoreType.DMA,
        pltpu.SemaphoreType.DMA,
    ],
    compiler_params=pltpu.CompilerParams(needs_layout_passes=False),
)
def _sc_merge(f1_hbm, f2_hbm, tb_hbm, out_hbm, fb, hist, tbv, ob, sem_in, sem_out):
    wid = lax.axis_index("s") * NC + lax.axis_index("c")
    ebase = wid * RPW * L
    pltpu.sync_copy(tb_hbm, tbv)
    iota = lax.iota(jnp.int32, 16)
    ones = jnp.ones((16,), jnp.float32)
    zeros = jnp.zeros((16,), jnp.float32)

    def start_in(c, slot):
        off = ebase + c * EPC
        return (
            pltpu.async_copy(f1_hbm.at[pl.ds(off, EPC)], fb.at[slot, 0], sem_in),
            pltpu.async_copy(f2_hbm.at[pl.ds(off, EPC)], fb.at[slot, 1], sem_in),
        )

    in_h = {0: start_in(0, 0)}
    out_h = [None, None]
    for c in range(NCHUNK):
        slot = c & 1
        if c + 1 < NCHUNK:
            in_h[c + 1] = start_in(c + 1, 1 - slot)
        cp1, cp2 = in_h.pop(c)
        cp1.wait()
        cp2.wait()

        def zero_body(i, _):
            hist[pl.ds(i * 16, 16)] = zeros
            return 0

        lax.fori_loop(0, 2 * CH, zero_body, 0, unroll=8)

        tailones = jnp.where(iota >= 8, 1.0, 0.0).astype(jnp.float32)

        def p1row(r, _, slot=slot):
            bvec = jnp.full((16,), 0, jnp.int32) + r * 32
            bvec16 = bvec + 16
            ebase = r * L
            starts = [k * 16 for k in range(12)] + [184]
            idx = []
            for s in starts:
                v1 = fb[slot, 0, pl.ds(ebase + s, 16)]
                v2 = fb[slot, 1, pl.ds(ebase + s, 16)]
                idx.append((bvec + v1, bvec16 + v2))
            for i1, i2 in idx[:-1]:
                plsc.addupdate_scatter(hist, [i1], ones)
                plsc.addupdate_scatter(hist, [i2], ones)
            i1, i2 = idx[-1]
            plsc.addupdate_scatter(hist, [i1], tailones)
            plsc.addupdate_scatter(hist, [i2], tailones)
            return 0

        lax.fori_loop(0, CH, p1row, 0)

        if out_h[slot] is not None:
            out_h[slot].wait()

        def p2(g, _, slot=slot):
            rows = g * 16 + iota
            rbins = rows * 32
            acc = [zeros, zeros, zeros]
            for v in range(VOCAB):
                c1 = plsc.load_gather(hist, [rbins + v])
                c2 = plsc.load_gather(hist, [rbins + (16 + v)])
                for d in range(DIM):
                    acc[d] = acc[d] + c1 * tbv[v * DIM + d] + c2 * tbv[(VOCAB + v) * DIM + d]
            rows3 = rows * 3 + slot * (CH * DIM)
            for d in range(DIM):
                plsc.store_scatter(ob, [rows3 + d], acc[d])
            return 0

        lax.fori_loop(0, CH // 16, p2, 0)

        out_h[slot] = pltpu.async_copy(
            ob.at[pl.ds(slot * CH * DIM, CH * DIM)],
            out_hbm.at[pl.ds((wid * RPW + c * CH) * DIM, CH * DIM)],
            sem_out,
        )

    for s in (0, 1):
        if out_h[s] is not None:
            out_h[s].wait()


def kernel(feature_1, feature_2, table_1, table_2):
    f1 = feature_1.reshape(-1)
    f2 = feature_2.reshape(-1)
    tb = jnp.concatenate([table_1.reshape(-1), table_2.reshape(-1)])
    tb = jnp.broadcast_to((tb * jnp.float32(1.0 / L))[:, None], (2 * VOCAB * DIM, 16))
    return _sc_merge(f1, f2, tb).reshape(B, DIM)


# speedup vs baseline: 1.0033x; 1.0033x over previous
import functools

import jax
import jax.numpy as jnp
from jax import lax
from jax.experimental import pallas as pl
from jax.experimental.pallas import tpu as pltpu
from jax.experimental.pallas import tpu_sc as plsc

B, L = 16384, 200
VOCAB, DIM = 10, 3
NC, NS = 2, 16
NW = NC * NS
RPW = B // NW
CH = 128
NCHUNK = RPW // CH
EPC = CH * L
VPC = EPC // 16
MAGIC = 5243

_mesh = plsc.VectorSubcoreMesh(core_axis_name="c", subcore_axis_name="s")


@functools.partial(
    pl.kernel,
    mesh=_mesh,
    out_type=jax.ShapeDtypeStruct((B * DIM,), jnp.float32),
    scratch_types=[
        pltpu.VMEM((2, 2, EPC), jnp.int32),
        pltpu.VMEM((2 * CH * 16,), jnp.float32),
        pltpu.VMEM((2 * VOCAB * DIM, 16), jnp.float32),
        pltpu.VMEM((2 * CH * DIM,), jnp.float32),
        pltpu.SemaphoreType.DMA,
        pltpu.SemaphoreType.DMA,
        pltpu.SemaphoreType.DMA,
        pltpu.SemaphoreType.DMA,
        pltpu.SemaphoreType.DMA,
    ],
    compiler_params=pltpu.CompilerParams(needs_layout_passes=False),
)
def _sc_merge(
    f1_hbm, f2_hbm, tb_hbm, out_hbm, fb, hist, tbv, ob,
    sem_s0f1, sem_s0f2, sem_s1f1, sem_s1f2, sem_out,
):
    sem_in = ((sem_s0f1, sem_s0f2), (sem_s1f1, sem_s1f2))
    wid = lax.axis_index("s") * NC + lax.axis_index("c")
    ebase = wid * RPW * L
    pltpu.sync_copy(tb_hbm, tbv)
    iota = lax.iota(jnp.int32, 16)
    ones = jnp.ones((16,), jnp.float32)
    zeros = jnp.zeros((16,), jnp.float32)

    def start_in(c, slot):
        off = ebase + c * EPC
        return (
            pltpu.async_copy(f1_hbm.at[pl.ds(off, EPC)], fb.at[slot, 0], sem_in[slot][0]),
            pltpu.async_copy(f2_hbm.at[pl.ds(off, EPC)], fb.at[slot, 1], sem_in[slot][1]),
        )

    in_h = {0: start_in(0, 0)}
    out_h = [None, None]
    for c in range(NCHUNK):
        slot = c & 1
        if c + 1 < NCHUNK:
            in_h[c + 1] = start_in(c + 1, 1 - slot)
        cp1, cp2 = in_h.pop(c)
        cp1.wait()
        cp2.wait()

        def zero_body(i, _):
            hist[pl.ds(i * 16, 16)] = zeros
            return 0

        lax.fori_loop(0, 2 * CH, zero_body, 0, unroll=8)

        straddle = (iota >> 3) * 32

        def p1pair(r2, _, slot=slot):
            bvec = jnp.full((16,), 0, jnp.int32) + r2 * 64
            ebase = r2 * (2 * L)
            offs = [bvec] * 12 + [bvec + straddle] + [bvec + 32] * 12
            for blk in range(0, 25, 5):
                idx = []
                for k in range(blk, min(blk + 5, 25)):
                    v1 = fb[slot, 0, pl.ds(ebase + k * 16, 16)]
                    v2 = fb[slot, 1, pl.ds(ebase + k * 16, 16)]
                    idx.append((offs[k] + v1, offs[k] + 16 + v2))
                for i1, i2 in idx:
                    plsc.addupdate_scatter(hist, [i1], ones)
                    plsc.addupdate_scatter(hist, [i2], ones)
            return 0

        lax.fori_loop(0, CH // 2, p1pair, 0)

        if out_h[slot] is not None:
            out_h[slot].wait()

        def p2(g, _, slot=slot):
            rows = g * 16 + iota
            rbins = rows * 32
            acc = [zeros, zeros, zeros]
            for v in range(VOCAB):
                c1 = plsc.load_gather(hist, [rbins + v])
                c2 = plsc.load_gather(hist, [rbins + (16 + v)])
                for d in range(DIM):
                    acc[d] = acc[d] + c1 * tbv[v * DIM + d] + c2 * tbv[(VOCAB + v) * DIM + d]
            rows3 = rows * 3 + slot * (CH * DIM)
            for d in range(DIM):
                plsc.store_scatter(ob, [rows3 + d], acc[d])
            return 0

        lax.fori_loop(0, CH // 16, p2, 0)

        out_h[slot] = pltpu.async_copy(
            ob.at[pl.ds(slot * CH * DIM, CH * DIM)],
            out_hbm.at[pl.ds((wid * RPW + c * CH) * DIM, CH * DIM)],
            sem_out,
        )

    for s in (0, 1):
        if out_h[s] is not None:
            out_h[s].wait()


def kernel(feature_1, feature_2, table_1, table_2):
    f1 = feature_1.reshape(-1)
    f2 = feature_2.reshape(-1)
    tb = jnp.concatenate([table_1.reshape(-1), table_2.reshape(-1)])
    tb = jnp.broadcast_to((tb * jnp.float32(1.0 / L))[:, None], (2 * VOCAB * DIM, 16))
    return _sc_merge(f1, f2, tb).reshape(B, DIM)


# speedup vs baseline: 1.0104x; 1.0071x over previous
import functools

import jax
import jax.numpy as jnp
from jax import lax
from jax.experimental import pallas as pl
from jax.experimental.pallas import tpu as pltpu
from jax.experimental.pallas import tpu_sc as plsc

B, L = 16384, 200
VOCAB, DIM = 10, 3
NC, NS = 2, 16
NW = NC * NS
RPW = B // NW
CH = 128
NCHUNK = RPW // CH
EPC = CH * L

_mesh = plsc.VectorSubcoreMesh(core_axis_name="c", subcore_axis_name="s")


@functools.partial(
    pl.kernel,
    mesh=_mesh,
    out_type=jax.ShapeDtypeStruct((B * DIM,), jnp.float32),
    scratch_types=[
        pltpu.VMEM((2, 2, EPC), jnp.int32),
        pltpu.VMEM((2 * CH * 16,), jnp.float32),
        pltpu.VMEM((2 * VOCAB * DIM, 16), jnp.float32),
        pltpu.VMEM((CH * DIM,), jnp.float32),
        pltpu.SemaphoreType.DMA,
        pltpu.SemaphoreType.DMA,
        pltpu.SemaphoreType.DMA,
        pltpu.SemaphoreType.DMA,
    ],
    compiler_params=pltpu.CompilerParams(needs_layout_passes=False),
)
def _sc_merge(
    f1_hbm, f2_hbm, tb_hbm, out_hbm, fb, hist, tbv, ob,
    sem_s0f1, sem_s0f2, sem_s1f1, sem_s1f2,
):
    sem_in = ((sem_s0f1, sem_s0f2), (sem_s1f1, sem_s1f2))
    wid = lax.axis_index("s") * NC + lax.axis_index("c")
    ebase = wid * RPW * L
    pltpu.sync_copy(tb_hbm, tbv)
    iota = lax.iota(jnp.int32, 16)
    ones = jnp.ones((16,), jnp.float32)
    zeros = jnp.zeros((16,), jnp.float32)
    straddle = (iota >> 3) * 32

    def start_in(c, slot):
        off = ebase + c * EPC
        pltpu.async_copy(f1_hbm.at[pl.ds(off, EPC)], fb.at[slot, 0], sem_in[slot][0])
        pltpu.async_copy(f2_hbm.at[pl.ds(off, EPC)], fb.at[slot, 1], sem_in[slot][1])

    def wait_in(slot):
        pltpu.make_async_copy(
            f1_hbm.at[pl.ds(0, EPC)], fb.at[slot, 0], sem_in[slot][0]
        ).wait()
        pltpu.make_async_copy(
            f2_hbm.at[pl.ds(0, EPC)], fb.at[slot, 1], sem_in[slot][1]
        ).wait()

    start_in(0, 0)
    start_in(1, 1)

    def chunk2(c2, _):
        for sl in (0, 1):
            c = c2 * 2 + sl
            wait_in(sl)

            @plsc.parallel_loop(0, 2 * CH, unroll=8)
            def _zero(i):
                hist[pl.ds(i * 16, 16)] = zeros

            @plsc.parallel_loop(0, CH // 2)
            def _p1pair(r2, sl=sl):
                bvec = jnp.full((16,), 0, jnp.int32) + r2 * 64
                eoff = r2 * (2 * L)
                offs = [bvec] * 12 + [bvec + straddle] + [bvec + 32] * 12
                for blk in range(0, 25, 5):
                    idx = []
                    for k in range(blk, min(blk + 5, 25)):
                        v1 = fb[sl, 0, pl.ds(eoff + k * 16, 16)]
                        v2 = fb[sl, 1, pl.ds(eoff + k * 16, 16)]
                        idx.append((offs[k] + v1, offs[k] + 16 + v2))
                    for i1, i2 in idx:
                        plsc.addupdate_scatter(hist, [i1], ones)
                        plsc.addupdate_scatter(hist, [i2], ones)

            @pl.when(c2 == 0)
            def _():
                start_in(c + 2, sl)

            @plsc.parallel_loop(0, CH // 16)
            def _p2(g):
                rows = g * 16 + iota
                rbins = rows * 32
                acc = [zeros, zeros, zeros]
                for v in range(VOCAB):
                    c1 = plsc.load_gather(hist, [rbins + v])
                    c2v = plsc.load_gather(hist, [rbins + (16 + v)])
                    for d in range(DIM):
                        acc[d] = acc[d] + c1 * tbv[v * DIM + d] + c2v * tbv[(VOCAB + v) * DIM + d]
                rows3 = rows * 3
                for d in range(DIM):
                    plsc.store_scatter(ob, [rows3 + d], acc[d])

            pltpu.sync_copy(
                ob, out_hbm.at[pl.ds((wid * RPW + c * CH) * DIM, CH * DIM)]
            )
        return 0

    lax.fori_loop(0, NCHUNK // 2, chunk2, 0)


def kernel(feature_1, feature_2, table_1, table_2):
    f1 = feature_1.reshape(-1)
    f2 = feature_2.reshape(-1)
    tb = jnp.concatenate([table_1.reshape(-1), table_2.reshape(-1)])
    tb = jnp.broadcast_to((tb * jnp.float32(1.0 / L))[:, None], (2 * VOCAB * DIM, 16))
    return _sc_merge(f1, f2, tb).reshape(B, DIM)


# speedup vs baseline: 1.1373x; 1.1256x over previous
import functools

import jax
import jax.numpy as jnp
from jax import lax
from jax.experimental import pallas as pl
from jax.experimental.pallas import tpu as pltpu
from jax.experimental.pallas import tpu_sc as plsc

B, L = 16384, 200
VOCAB, DIM = 10, 3
NC, NS = 2, 16
NW = NC * NS
RPW = B // NW
CH = 64
NCHUNK = RPW // CH
LT = 192
TVR = CH * 8 // 128

_mesh = plsc.VectorSubcoreMesh(core_axis_name="c", subcore_axis_name="s")


@functools.partial(
    pl.kernel,
    mesh=_mesh,
    out_type=jax.ShapeDtypeStruct((B * DIM,), jnp.float32),
    scratch_types=[
        pltpu.VMEM((2, 2, CH, L), jnp.int32),
        pltpu.VMEM((2, 2, TVR, 128), jnp.int32),
        pltpu.VMEM((2 * CH * 16,), jnp.float32),
        pltpu.VMEM((2 * VOCAB * DIM, 16), jnp.float32),
        pltpu.VMEM((CH * DIM,), jnp.float32),
        pltpu.SemaphoreType.DMA,
        pltpu.SemaphoreType.DMA,
        pltpu.SemaphoreType.DMA,
        pltpu.SemaphoreType.DMA,
    ],
    compiler_params=pltpu.CompilerParams(needs_layout_passes=False),
)
def _sc_merge(
    f1_hbm, f2_hbm, t1_hbm, t2_hbm, tb_hbm, out_hbm, fb, tbuf, hist, tbv, ob,
    sem_s0f1, sem_s0f2, sem_s1f1, sem_s1f2,
):
    sem_in = ((sem_s0f1, sem_s0f2), (sem_s1f1, sem_s1f2))
    f_hbm = (f1_hbm, f2_hbm)
    t_hbm = (t1_hbm, t2_hbm)
    wid = lax.axis_index("s") * NC + lax.axis_index("c")
    pltpu.sync_copy(tb_hbm, tbv)
    iota = lax.iota(jnp.int32, 16)
    ones = jnp.ones((16,), jnp.float32)
    zeros = jnp.zeros((16,), jnp.float32)
    straddle = (iota >> 3) * 32

    def start_in(c, slot):
        row0 = pl.multiple_of(wid * RPW + c * CH, CH)
        trow0 = pl.multiple_of((wid * RPW + c * CH) // 16, TVR)
        for f in (0, 1):
            pltpu.async_copy(
                f_hbm[f].at[pl.ds(row0, CH)], fb.at[slot, f], sem_in[slot][f]
            )
            pltpu.async_copy(
                t_hbm[f].at[pl.ds(trow0, TVR)], tbuf.at[slot, f], sem_in[slot][f]
            )

    def wait_in(slot):
        for f in (0, 1):
            pltpu.make_async_copy(
                f_hbm[f].at[pl.ds(0, CH)], fb.at[slot, f], sem_in[slot][f]
            ).wait()
            pltpu.make_async_copy(
                t_hbm[f].at[pl.ds(0, TVR)], tbuf.at[slot, f], sem_in[slot][f]
            ).wait()

    start_in(0, 0)
    start_in(1, 1)

    def chunk2(c2, _):
        for sl in (0, 1):
            c = c2 * 2 + sl
            wait_in(sl)

            @plsc.parallel_loop(0, 2 * CH, unroll=8)
            def _zero(i):
                hist[pl.ds(i * 16, 16)] = zeros

            @plsc.parallel_loop(0, CH)
            def _p1row(r, sl=sl):
                bvec = jnp.full((16,), 0, jnp.int32) + r * 32
                for blk in range(0, 12, 4):
                    idx = []
                    for k in range(blk, blk + 4):
                        v1 = fb[sl, 0, r, pl.ds(k * 16, 16)]
                        v2 = fb[sl, 1, r, pl.ds(k * 16, 16)]
                        idx.append((bvec + v1, bvec + 16 + v2))
                    for i1, i2 in idx:
                        plsc.addupdate_scatter(hist, [i1], ones)
                        plsc.addupdate_scatter(hist, [i2], ones)

            @plsc.parallel_loop(0, CH // 2)
            def _p1tail(k, sl=sl):
                off = straddle + k * 64
                vr = k >> 3
                ls = (k & 7) * 16
                v1 = tbuf[sl, 0, vr, pl.ds(ls, 16)]
                v2 = tbuf[sl, 1, vr, pl.ds(ls, 16)]
                plsc.addupdate_scatter(hist, [off + v1], ones)
                plsc.addupdate_scatter(hist, [off + 16 + v2], ones)

            @pl.when(c2 < (NCHUNK // 2 - 1))
            def _():
                start_in(c + 2, sl)

            @plsc.parallel_loop(0, CH // 16)
            def _p2(g):
                rows = g * 16 + iota
                rbins = rows * 32
                acc = [zeros, zeros, zeros]
                for v in range(VOCAB):
                    c1 = plsc.load_gather(hist, [rbins + v])
                    c2v = plsc.load_gather(hist, [rbins + (16 + v)])
                    for d in range(DIM):
                        acc[d] = acc[d] + c1 * tbv[v * DIM + d] + c2v * tbv[(VOCAB + v) * DIM + d]
                rows3 = rows * 3
                for d in range(DIM):
                    plsc.store_scatter(ob, [rows3 + d], acc[d])

            pltpu.sync_copy(
                ob, out_hbm.at[pl.ds((wid * RPW + c * CH) * DIM, CH * DIM)]
            )
        return 0

    lax.fori_loop(0, NCHUNK // 2, chunk2, 0)


def kernel(feature_1, feature_2, table_1, table_2):
    t1 = feature_1[:, LT:].reshape(B * (L - LT) // 128, 128)
    t2 = feature_2[:, LT:].reshape(B * (L - LT) // 128, 128)
    tb = jnp.concatenate([table_1.reshape(-1), table_2.reshape(-1)])
    tb = jnp.broadcast_to((tb * jnp.float32(1.0 / L))[:, None], (2 * VOCAB * DIM, 16))
    return _sc_merge(feature_1, feature_2, t1, t2, tb).reshape(B, DIM)
